# baseline (device time: 105399 ns/iter reference)
import jax
import jax.numpy as jnp
from jax import lax
from jax.experimental import pallas as pl
from jax.experimental.pallas import tpu as pltpu

N_DEV = 16
NSUB = 2
N_HOPS = N_DEV - 1
MM_ROWS = 1024


def kernel(x, w_mat, scale_x, scale_w):
    k, n = w_mat.shape
    m = x.shape[0]
    m_per = m // N_DEV
    n_streams = 2 * NSUB
    ns = n // n_streams

    def body(x_ref, w_ref, sx_ref, sw_ref, out_ref, *scratch):
        p_ref = scratch[0]
        comm_bufs = scratch[1:1 + n_streams]
        send_sems = scratch[1 + n_streams:1 + 2 * n_streams]
        recv_sems = scratch[1 + 2 * n_streams:1 + 3 * n_streams]

        my = lax.axis_index("i")
        left = lax.rem(my + N_DEV - 1, N_DEV)
        right = lax.rem(my + 1, N_DEV)

        def tgt(st):
            return right if st < NSUB else left

        def send_chunk(d, s):
            off = N_DEV - 1 - s if d == 0 else 1 + s
            return lax.rem(my + 2 * N_DEV + off, N_DEV)

        def recv_chunk(d, s):
            off = N_DEV - 2 - s if d == 0 else 2 + s
            return lax.rem(my + 2 * N_DEV + off, N_DEV)

        def p_slice(st, c):
            return p_ref[pl.ds(c * m_per, m_per), st * ns:(st + 1) * ns]

        wb = w_ref[:, :].astype(jnp.bfloat16)
        for r in range(m // MM_ROWS):
            p_ref[r * MM_ROWS:(r + 1) * MM_ROWS, :] = jnp.dot(
                x_ref[r * MM_ROWS:(r + 1) * MM_ROWS, :].astype(jnp.bfloat16),
                wb, preferred_element_type=jnp.float32).astype(jnp.bfloat16)

        barrier_sem = pltpu.get_barrier_semaphore()
        for nbr in (left, right):
            pl.semaphore_signal(barrier_sem, inc=1, device_id=(nbr,),
                                device_id_type=pl.DeviceIdType.MESH)
        pl.semaphore_wait(barrier_sem, 2)

        def make_rdma(st, s):
            if s == 0:
                src = p_ref.at[pl.ds(send_chunk(st // NSUB, 0) * m_per, m_per),
                               pl.ds(st * ns, ns)]
            else:
                src = comm_bufs[st].at[s - 1]
            return pltpu.make_async_remote_copy(
                src_ref=src,
                dst_ref=comm_bufs[st].at[s],
                send_sem=send_sems[st].at[s],
                recv_sem=recv_sems[st].at[s],
                device_id=(tgt(st),),
                device_id_type=pl.DeviceIdType.MESH,
            )

        rdmas = [[None] * N_HOPS for _ in range(n_streams)]
        for st in range(n_streams):
            rdmas[st][0] = make_rdma(st, 0)
            rdmas[st][0].start()

        order = []
        for j in range(NSUB):
            order += [j, NSUB + j]

        scale = sx_ref[0] * sw_ref[0]
        for s in range(N_HOPS):
            for st in order:
                rdmas[st][s].wait_recv()
                c = recv_chunk(st // NSUB, s)
                if s < N_HOPS - 1:
                    comm_bufs[st][s, :, :] = (
                        comm_bufs[st][s, :, :] + p_slice(st, c))
                    rdmas[st][s + 1] = make_rdma(st, s + 1)
                    rdmas[st][s + 1].start()
                else:
                    acc = (comm_bufs[st][s, :, :].astype(jnp.float32)
                           + p_slice(st, c).astype(jnp.float32))
                    out_ref[:, st * ns:(st + 1) * ns] = jnp.maximum(
                        acc * scale, 0.0)

        for st in range(n_streams):
            for s in range(N_HOPS):
                rdmas[st][s].wait_send()

    return pl.pallas_call(
        body,
        out_shape=jax.ShapeDtypeStruct((m_per, n), jnp.float32),
        in_specs=[
            pl.BlockSpec(memory_space=pltpu.VMEM),
            pl.BlockSpec(memory_space=pltpu.VMEM),
            pl.BlockSpec(memory_space=pltpu.SMEM),
            pl.BlockSpec(memory_space=pltpu.SMEM),
        ],
        out_specs=pl.BlockSpec(memory_space=pltpu.VMEM),
        scratch_shapes=(
            [pltpu.VMEM((m, n), jnp.bfloat16)] +
            [pltpu.VMEM((N_HOPS, m_per, ns), jnp.bfloat16)] * n_streams +
            [pltpu.SemaphoreType.DMA((N_HOPS,))] * n_streams +
            [pltpu.SemaphoreType.DMA((N_HOPS,))] * n_streams
        ),
        compiler_params=pltpu.CompilerParams(
            collective_id=0, vmem_limit_bytes=100 * 1024 * 1024),
    )(x, w_mat, scale_x, scale_w)


# device time: 98727 ns/iter; 1.0676x vs baseline; 1.0676x over previous
import jax
import jax.numpy as jnp
from jax import lax
from jax.experimental import pallas as pl
from jax.experimental.pallas import tpu as pltpu

N_DEV = 16
NSUB = 2
N_HOPS = N_DEV - 1


def kernel(x, w_mat, scale_x, scale_w):
    k, n = w_mat.shape
    m = x.shape[0]
    m_per = m // N_DEV
    n_streams = 2 * NSUB
    ns = n // n_streams

    def body(x_ref, w_ref, sx_ref, sw_ref, out_ref, *scratch):
        send_bufs = scratch[0:n_streams]
        comm_bufs = scratch[n_streams:2 * n_streams]
        send_sems = scratch[2 * n_streams:3 * n_streams]
        recv_sems = scratch[3 * n_streams:4 * n_streams]

        my = lax.axis_index("i")
        left = lax.rem(my + N_DEV - 1, N_DEV)
        right = lax.rem(my + 1, N_DEV)

        def tgt(st):
            return right if st < NSUB else left

        def send_chunk(d, s):
            off = N_DEV - 1 - s if d == 0 else 1 + s
            return lax.rem(my + 2 * N_DEV + off, N_DEV)

        def recv_chunk(d, s):
            off = N_DEV - 2 - s if d == 0 else 2 + s
            return lax.rem(my + 2 * N_DEV + off, N_DEV)

        w_st = [w_ref[:, st * ns:(st + 1) * ns].astype(jnp.bfloat16)
                for st in range(n_streams)]

        def xblk(c):
            return x_ref[pl.ds(c * m_per, m_per), :].astype(jnp.bfloat16)

        def partials(s):
            xa = xblk(recv_chunk(0, s))
            xb = xblk(recv_chunk(1, s))
            return [jnp.dot(xa if st < NSUB else xb, w_st[st],
                            preferred_element_type=jnp.float32)
                    for st in range(n_streams)]

        xa0 = xblk(send_chunk(0, 0))
        xb0 = xblk(send_chunk(1, 0))
        for st in range(n_streams):
            send_bufs[st][0, :, :] = jnp.dot(
                xa0 if st < NSUB else xb0, w_st[st],
                preferred_element_type=jnp.float32).astype(jnp.bfloat16)

        barrier_sem = pltpu.get_barrier_semaphore()
        for nbr in (left, right):
            pl.semaphore_signal(barrier_sem, inc=1, device_id=(nbr,),
                                device_id_type=pl.DeviceIdType.MESH)
        pl.semaphore_wait(barrier_sem, 2)

        def make_rdma(st, s):
            return pltpu.make_async_remote_copy(
                src_ref=send_bufs[st].at[s % 2],
                dst_ref=comm_bufs[st].at[s],
                send_sem=send_sems[st].at[s],
                recv_sem=recv_sems[st].at[s],
                device_id=(tgt(st),),
                device_id_type=pl.DeviceIdType.MESH,
            )

        rdmas = [[None] * N_HOPS for _ in range(n_streams)]
        for st in range(n_streams):
            rdmas[st][0] = make_rdma(st, 0)
            rdmas[st][0].start()

        p = partials(0)

        order = []
        for j in range(NSUB):
            order += [j, NSUB + j]

        scale = sx_ref[0] * sw_ref[0]
        for s in range(N_HOPS):
            for st in order:
                rdmas[st][s].wait_recv()
                acc = comm_bufs[st][s, :, :].astype(jnp.float32) + p[st]
                if s < N_HOPS - 1:
                    if s >= 1:
                        rdmas[st][s - 1].wait_send()
                    send_bufs[st][(s + 1) % 2, :, :] = acc.astype(jnp.bfloat16)
                    rdmas[st][s + 1] = make_rdma(st, s + 1)
                    rdmas[st][s + 1].start()
                else:
                    out_ref[:, st * ns:(st + 1) * ns] = jnp.maximum(
                        acc * scale, 0.0)
            if s < N_HOPS - 1:
                p = partials(s + 1)

        for st in range(n_streams):
            rdmas[st][N_HOPS - 2].wait_send()
            rdmas[st][N_HOPS - 1].wait_send()

    return pl.pallas_call(
        body,
        out_shape=jax.ShapeDtypeStruct((m_per, n), jnp.float32),
        in_specs=[
            pl.BlockSpec(memory_space=pltpu.VMEM),
            pl.BlockSpec(memory_space=pltpu.VMEM),
            pl.BlockSpec(memory_space=pltpu.SMEM),
            pl.BlockSpec(memory_space=pltpu.SMEM),
        ],
        out_specs=pl.BlockSpec(memory_space=pltpu.VMEM),
        scratch_shapes=(
            [pltpu.VMEM((2, m_per, ns), jnp.bfloat16)] * n_streams +
            [pltpu.VMEM((N_HOPS, m_per, ns), jnp.bfloat16)] * n_streams +
            [pltpu.SemaphoreType.DMA((N_HOPS,))] * n_streams +
            [pltpu.SemaphoreType.DMA((N_HOPS,))] * n_streams
        ),
        compiler_params=pltpu.CompilerParams(collective_id=0),
    )(x, w_mat, scale_x, scale_w)
